# baseline (speedup 1.0000x reference)
import math

import jax
import jax.numpy as jnp
from jax import lax
from jax.experimental import pallas as pl
from jax.experimental.pallas import tpu as pltpu
from jax.experimental.pallas import tpu_sc as plsc

D_MODEL = 64
SCALE = math.sqrt(D_MODEL)

NUM_CORES = 2
NUM_SUBCORES = 16
NUM_WORKERS = NUM_CORES * NUM_SUBCORES

VOCAB = 1000000
LANES = 128
N_TCOL = (VOCAB + LANES - 1) // LANES
KCOL = 2
N_SB = (N_TCOL + KCOL - 1) // KCOL
SB_PER_WORKER = (N_SB + NUM_WORKERS - 1) // NUM_WORKERS
N_PAIR = N_SB * KCOL * (LANES // 2)

B_TOTAL = 4096 * 50
ROWS_PER_WORKER = B_TOTAL // NUM_WORKERS
CHUNK = 256
NUM_CHUNKS = ROWS_PER_WORKER // CHUNK
SUBGATHER = 128
NUM_SUB = CHUNK // SUBGATHER
ROW_UNROLL = 4
D_PAIR = 2 * D_MODEL


def _relayout_kernel(lut_t, tail_t, out_hbm,
                     buf0, buf1, buf2, pair0, pair1, pair2,
                     rsem0, rsem1, rsem2, wsem0, wsem1, wsem2):
    wid = lax.axis_index("s") * NUM_CORES + lax.axis_index("c")
    sb0 = wid * SB_PER_WORKER
    n_my = jnp.minimum(SB_PER_WORKER, jnp.maximum(N_SB - sb0, 0))
    bufs = (buf0, buf1, buf2)
    pairs = (pair0, pair1, pair2)
    rsems = (rsem0, rsem1, rsem2)
    wsems = (wsem0, wsem1, wsem2)
    W = KCOL * LANES
    NP = W // 2

    iota16 = lax.iota(jnp.int32, 16)
    pidx_base = iota16 // 2
    cidx_base = (iota16 % 2) * 64

    def fire(t, b):
        sb = sb0 + t

        @pl.when(sb < N_SB - 1)
        def _():
            pltpu.async_copy(
                lut_t.at[:, pl.ds(sb * W, W)], bufs[b], rsems[b])

        @pl.when(sb == N_SB - 1)
        def _():
            pltpu.async_copy(
                tail_t, bufs[b].at[:, pl.ds(0, LANES)], rsems[b])

    def drain(t, b):
        sb = sb0 + t

        @pl.when(sb < N_SB - 1)
        def _():
            pltpu.make_async_copy(
                lut_t.at[:, pl.ds(0, W)], bufs[b], rsems[b]).wait()

        @pl.when(sb == N_SB - 1)
        def _():
            pltpu.make_async_copy(
                tail_t, bufs[b].at[:, pl.ds(0, LANES)], rsems[b]).wait()

    def step(t, b):
        @pl.when(t + 2 < n_my)
        def _():
            fire(t + 2, (b + 2) % 3)

        drain(t, b)
        sb = sb0 + t
        buf = bufs[b]
        pair = pairs[b]

        @pl.when(t >= 3)
        def _():
            pltpu.make_async_copy(
                pair, out_hbm.at[pl.ds(0, NP)], wsems[b]).wait()

        @plsc.parallel_loop(0, W // 16, unroll=2)
        def _(m):
            pidx = pidx_base + 8 * m
            for dd in range(64):
                vals = buf[dd, pl.ds(16 * m, 16)]
                plsc.store_scatter(pair, [pidx, cidx_base + dd], vals)

        pltpu.async_copy(pair, out_hbm.at[pl.ds(sb * NP, NP)], wsems[b])

    def super_body(t2, _):
        for b in range(3):
            t = 3 * t2 + b

            @pl.when(t < n_my)
            def _(t=t, b=b):
                step(t, b)
        return None

    @pl.when(n_my > 0)
    def _():
        fire(0, 0)

        @pl.when(n_my > 1)
        def _():
            fire(1, 1)

        lax.fori_loop(0, (SB_PER_WORKER + 2) // 3, super_body, None)
        for b in range(3):
            @pl.when(n_my > b)
            def _(b=b):
                pltpu.make_async_copy(
                    pairs[b], out_hbm.at[pl.ds(0, NP)], wsems[b]).wait()


def _gather_kernel(lut_hbm, idx_hbm, out_hbm,
                   idx0, idx1, rows0, rows1, gsem0, gsem1, wsem0, wsem1):
    wid = lax.axis_index("s") * NUM_CORES + lax.axis_index("c")
    base = wid * ROWS_PER_WORKER
    idx_bufs = (idx0, idx1)
    row_bufs = (rows0, rows1)
    gsems = (gsem0, gsem1)
    wsems = (wsem0, wsem1)

    def fire(c):
        b = c % 2
        row0 = base + c * CHUNK
        pltpu.sync_copy(idx_hbm.at[pl.ds(row0, CHUNK)], idx_bufs[b])
        cps = []
        for j in range(NUM_SUB):
            cps.append(pltpu.async_copy(
                lut_hbm.at[idx_bufs[b].at[pl.ds(j * SUBGATHER, SUBGATHER)]],
                row_bufs[b].at[pl.ds(j * SUBGATHER, SUBGATHER)],
                gsems[b],
            ))
        return cps

    def write(c):
        b = c % 2
        row0 = base + c * CHUNK
        return pltpu.async_copy(row_bufs[b], out_hbm.at[pl.ds(row0, CHUNK)],
                                wsems[b])

    pending_g = {0: fire(0)}
    pending_w = {}
    for c in range(NUM_CHUNKS):
        b = c % 2
        if c + 1 < NUM_CHUNKS:
            if c - 1 in pending_w:
                pending_w.pop(c - 1).wait()
            pending_g[c + 1] = fire(c + 1)
        for cp in pending_g.pop(c):
            cp.wait()

        rows = row_bufs[b]

        def mul_body(i, _):
            for rr in range(ROW_UNROLL):
                r = i * ROW_UNROLL + rr
                for k in range(D_PAIR // 16):
                    sl = pl.ds(k * 16, 16)
                    rows[r, sl] = rows[r, sl] * SCALE
            return None
        lax.fori_loop(0, CHUNK // ROW_UNROLL, mul_body, None)

        pending_w[c] = write(c)
    for cp in pending_w.values():
        cp.wait()


@jax.jit
def kernel(x, lut):
    idx = x.reshape(-1).astype(jnp.int32)
    ih = idx >> 1
    parity = (idx & 1).astype(jnp.int32)
    lut_t = lut.T
    tail_t = jnp.pad(lut_t[:, (N_TCOL - 1) * LANES:],
                     ((0, 0), (0, N_TCOL * LANES - VOCAB)))
    mesh = plsc.VectorSubcoreMesh(core_axis_name="c", subcore_axis_name="s")

    lut2 = pl.kernel(
        _relayout_kernel,
        mesh=mesh,
        compiler_params=pltpu.CompilerParams(use_tc_tiling_on_sc=True,
                                             needs_layout_passes=False,
                                             disable_bounds_checks=True),
        out_type=jax.ShapeDtypeStruct((N_PAIR, D_PAIR), jnp.float32),
        scratch_types=[
            pltpu.VMEM((64, KCOL * LANES), jnp.float32),
            pltpu.VMEM((64, KCOL * LANES), jnp.float32),
            pltpu.VMEM((64, KCOL * LANES), jnp.float32),
            pltpu.VMEM((KCOL * LANES // 2, D_PAIR), jnp.float32),
            pltpu.VMEM((KCOL * LANES // 2, D_PAIR), jnp.float32),
            pltpu.VMEM((KCOL * LANES // 2, D_PAIR), jnp.float32),
            pltpu.SemaphoreType.DMA,
            pltpu.SemaphoreType.DMA,
            pltpu.SemaphoreType.DMA,
            pltpu.SemaphoreType.DMA,
            pltpu.SemaphoreType.DMA,
            pltpu.SemaphoreType.DMA,
        ],
    )(lut_t, tail_t)

    out = pl.kernel(
        _gather_kernel,
        mesh=mesh,
        compiler_params=pltpu.CompilerParams(use_tc_tiling_on_sc=True),
        out_type=jax.ShapeDtypeStruct((B_TOTAL, D_PAIR), jnp.float32),
        scratch_types=[
            pltpu.VMEM((CHUNK,), jnp.int32),
            pltpu.VMEM((CHUNK,), jnp.int32),
            pltpu.VMEM((CHUNK, D_PAIR), jnp.float32),
            pltpu.VMEM((CHUNK, D_PAIR), jnp.float32),
            pltpu.SemaphoreType.DMA,
            pltpu.SemaphoreType.DMA,
            pltpu.SemaphoreType.DMA,
            pltpu.SemaphoreType.DMA,
        ],
    )(lut2, ih)
    sel = jnp.where((parity == 1)[:, None], out[:, D_MODEL:], out[:, :D_MODEL])
    return sel.reshape(x.shape[0], x.shape[1], D_MODEL)


# speedup vs baseline: 1.8207x; 1.8207x over previous
import math

import jax
import jax.numpy as jnp
from jax import lax
from jax.experimental import pallas as pl
from jax.experimental.pallas import tpu as pltpu
from jax.experimental.pallas import tpu_sc as plsc

D_MODEL = 64
SCALE = math.sqrt(D_MODEL)

NUM_CORES = 2
NUM_SUBCORES = 16
NUM_WORKERS = NUM_CORES * NUM_SUBCORES

B_TOTAL = 4096 * 50
ROWS_PER_WORKER = B_TOTAL // NUM_WORKERS
CHUNK = 256
NUM_CHUNKS = ROWS_PER_WORKER // CHUNK
SUBGATHER = 128
NUM_SUB = CHUNK // SUBGATHER
ROW_UNROLL = 4
D_PAD = 128


def _emb_kernel(lut_hbm, idx_hbm, out_hbm,
                idx0, idx1, rows0, rows1, gsem0, gsem1, wsem0, wsem1):
    wid = lax.axis_index("s") * NUM_CORES + lax.axis_index("c")
    base = wid * ROWS_PER_WORKER
    idx_bufs = (idx0, idx1)
    row_bufs = (rows0, rows1)
    gsems = (gsem0, gsem1)
    wsems = (wsem0, wsem1)

    def fire(c):
        b = c % 2
        row0 = base + c * CHUNK
        pltpu.sync_copy(idx_hbm.at[pl.ds(row0, CHUNK)], idx_bufs[b])
        cps = []
        for j in range(NUM_SUB):
            cps.append(pltpu.async_copy(
                lut_hbm.at[idx_bufs[b].at[pl.ds(j * SUBGATHER, SUBGATHER)]],
                row_bufs[b].at[pl.ds(j * SUBGATHER, SUBGATHER)],
                gsems[b],
            ))
        return cps

    def write(c):
        b = c % 2
        row0 = base + c * CHUNK
        return pltpu.async_copy(row_bufs[b], out_hbm.at[pl.ds(row0, CHUNK)],
                                wsems[b])

    pending_g = {0: fire(0)}
    pending_w = {}
    for c in range(NUM_CHUNKS):
        b = c % 2
        if c + 1 < NUM_CHUNKS:
            if c - 1 in pending_w:
                pending_w.pop(c - 1).wait()
            pending_g[c + 1] = fire(c + 1)
        for cp in pending_g.pop(c):
            cp.wait()

        rows = row_bufs[b]

        def mul_body(i, _):
            for rr in range(ROW_UNROLL):
                r = i * ROW_UNROLL + rr
                for k in range(D_MODEL // 16):
                    sl = pl.ds(k * 16, 16)
                    rows[r, sl] = rows[r, sl] * SCALE
            return None
        lax.fori_loop(0, CHUNK // ROW_UNROLL, mul_body, None)

        pending_w[c] = write(c)
    for cp in pending_w.values():
        cp.wait()


@jax.jit
def kernel(x, lut):
    idx = x.reshape(-1).astype(jnp.int32)
    lutp = jnp.pad(lut, ((0, 0), (0, D_PAD - D_MODEL)))
    mesh = plsc.VectorSubcoreMesh(core_axis_name="c", subcore_axis_name="s")
    out = pl.kernel(
        _emb_kernel,
        mesh=mesh,
        compiler_params=pltpu.CompilerParams(use_tc_tiling_on_sc=True),
        out_type=jax.ShapeDtypeStruct((B_TOTAL, D_PAD), jnp.float32),
        scratch_types=[
            pltpu.VMEM((CHUNK,), jnp.int32),
            pltpu.VMEM((CHUNK,), jnp.int32),
            pltpu.VMEM((CHUNK, D_PAD), jnp.float32),
            pltpu.VMEM((CHUNK, D_PAD), jnp.float32),
            pltpu.SemaphoreType.DMA,
            pltpu.SemaphoreType.DMA,
            pltpu.SemaphoreType.DMA,
            pltpu.SemaphoreType.DMA,
        ],
    )(lutp, idx)
    return out[:, :D_MODEL].reshape(x.shape[0], x.shape[1], D_MODEL)
